# baseline (speedup 1.0000x reference)
import functools

import jax
import jax.numpy as jnp
from jax import lax
from jax.experimental import pallas as pl
from jax.experimental.pallas import tpu as pltpu
from jax.experimental.pallas import tpu_sc as plsc

NUM_CORES = 2
NUM_SUBCORES = 16
NUM_WORKERS = NUM_CORES * NUM_SUBCORES

BATCH = 16384
HIST_LEN = 50
DIM = 64
VOCAB = 1000000
ROWS_PER_W = BATCH // NUM_WORKERS
NBUF = 8
N_GROUPS = ROWS_PER_W // NBUF

_MESH = plsc.VectorSubcoreMesh(
    core_axis_name="c",
    subcore_axis_name="s",
    num_cores=NUM_CORES,
    num_subcores=NUM_SUBCORES,
)


@functools.partial(
    pl.kernel,
    out_type=jax.ShapeDtypeStruct((BATCH, HIST_LEN, DIM), jnp.float32),
    mesh=_MESH,
    scratch_types=(
        [pltpu.VMEM((ROWS_PER_W, HIST_LEN), jnp.int32)]
        + [pltpu.VMEM((1, HIST_LEN, DIM), jnp.float32) for _ in range(NBUF)]
        + [pltpu.SemaphoreType.DMA for _ in range(2 * NBUF)]
    ),
    compiler_params=pltpu.CompilerParams(use_tc_tiling_on_sc=False),
)
def _gather_kernel(table_hbm, idx_hbm, out_hbm, idx_all, *bufs):
    rows = list(bufs[:NBUF])
    sem_g = list(bufs[NBUF:2 * NBUF])
    sem_w = list(bufs[2 * NBUF:])

    wid = lax.axis_index("s") * NUM_CORES + lax.axis_index("c")
    base = wid * ROWS_PER_W

    pltpu.sync_copy(idx_hbm.at[pl.ds(base, ROWS_PER_W)], idx_all)

    def gather_desc(j, b):
        return pltpu.make_async_copy(
            table_hbm.at[idx_all.at[pl.ds(j, 1)]], rows[b], sem_g[b])

    def write_desc(j, b):
        return pltpu.make_async_copy(
            rows[b], out_hbm.at[pl.ds(base + j, 1)], sem_w[b])

    for b in range(NBUF - 1):
        gather_desc(b, b).start()

    def group(g, _):
        for b in range(NBUF):
            j = g * NBUF + b
            bn = (b + NBUF - 1) % NBUF
            jn = j + NBUF - 1

            @pl.when(jn < ROWS_PER_W)
            def _():
                @pl.when(jn >= NBUF)
                def _():
                    write_desc(jn - NBUF, bn).wait()

                gather_desc(jn, bn).start()

            gather_desc(j, b).wait()
            write_desc(j, b).start()
        return 0

    lax.fori_loop(0, N_GROUPS, group, 0)

    for b in range(NBUF):
        write_desc(ROWS_PER_W - NBUF + b, b).wait()


def kernel(x, embedding):
    return _gather_kernel(embedding.reshape(1, VOCAB, DIM), x)
